# baseline (speedup 1.0000x reference)
import functools

import jax
import jax.numpy as jnp
from jax import lax
from jax.experimental import pallas as pl
from jax.experimental.pallas import tpu as pltpu
from jax.experimental.pallas import tpu_sc as plsc

_NC = 2
_NS = 16
_NW = _NC * _NS


_CHUNK = 128


@functools.cache
def _gather_fn(B, D):
    b_per_w = B // _NW
    n_chunks = b_per_w // _CHUNK
    mesh = plsc.VectorSubcoreMesh(core_axis_name="c", subcore_axis_name="s")

    @functools.partial(
        pl.kernel,
        out_type=[
            jax.ShapeDtypeStruct((B, D), jnp.float32),
            jax.ShapeDtypeStruct((B, D), jnp.float32),
        ],
        mesh=mesh,
        scratch_types=[
            pltpu.VMEM((n_chunks, _CHUNK), jnp.int32),
            pltpu.VMEM((b_per_w, D), jnp.float32),
            pltpu.VMEM((n_chunks, _CHUNK), jnp.int32),
            pltpu.VMEM((b_per_w, D), jnp.float32),
            pltpu.SemaphoreType.DMA,
        ],
        compiler_params=pltpu.CompilerParams(use_tc_tiling_on_sc=False),
    )
    def gather(uids_hbm, utab_hbm, iids_hbm, itab_hbm, uout_hbm, iout_hbm,
               uidx_v, urows_v, iidx_v, irows_v, sem):
        wid = lax.axis_index("s") * _NC + lax.axis_index("c")
        base = wid * b_per_w
        copies = []
        for j in range(n_chunks):
            pltpu.sync_copy(uids_hbm.at[pl.ds(base + j * _CHUNK, _CHUNK)],
                            uidx_v.at[j])
            pltpu.sync_copy(iids_hbm.at[pl.ds(base + j * _CHUNK, _CHUNK)],
                            iidx_v.at[j])
        for j in range(n_chunks):
            copies.append(pltpu.async_copy(
                utab_hbm.at[uidx_v.at[j]],
                urows_v.at[pl.ds(j * _CHUNK, _CHUNK)], sem))
            copies.append(pltpu.async_copy(
                itab_hbm.at[iidx_v.at[j]],
                irows_v.at[pl.ds(j * _CHUNK, _CHUNK)], sem))
        for cp in copies:
            cp.wait()
        pltpu.sync_copy(urows_v, uout_hbm.at[pl.ds(base, b_per_w)])
        pltpu.sync_copy(irows_v, iout_hbm.at[pl.ds(base, b_per_w)])

    return gather


def _mlp_body(ue_ref, ie_ref, w1u_ref, w1i_ref, b1_ref, w2_ref, b2_ref,
              wo_ref, bo_ref, out_ref):
    x1 = jnp.dot(ue_ref[...], w1u_ref[...], preferred_element_type=jnp.float32)
    x2 = jnp.dot(ie_ref[...], w1i_ref[...], preferred_element_type=jnp.float32)
    h = jnp.maximum(x1 + x2 + b1_ref[...], 0.0)
    h = jnp.maximum(
        jnp.dot(h, w2_ref[...], preferred_element_type=jnp.float32)
        + b2_ref[...], 0.0)
    out_ref[...] = jnp.sum(h * wo_ref[...], axis=1) + bo_ref[0]


@functools.cache
def _mlp_fn(B, D, H1, H2, bb):
    grid = B // bb
    return pl.pallas_call(
        _mlp_body,
        grid=(grid,),
        in_specs=[
            pl.BlockSpec((bb, D), lambda i: (i, 0)),
            pl.BlockSpec((bb, D), lambda i: (i, 0)),
            pl.BlockSpec((D, H1), lambda i: (0, 0)),
            pl.BlockSpec((D, H1), lambda i: (0, 0)),
            pl.BlockSpec((1, H1), lambda i: (0, 0)),
            pl.BlockSpec((H1, H2), lambda i: (0, 0)),
            pl.BlockSpec((1, H2), lambda i: (0, 0)),
            pl.BlockSpec((1, H2), lambda i: (0, 0)),
            pl.BlockSpec((1,), lambda i: (0,)),
        ],
        out_specs=pl.BlockSpec((bb,), lambda i: (i,)),
        out_shape=jax.ShapeDtypeStruct((B,), jnp.float32),
    )


def kernel(user_ids, item_ids, user_table, item_table, W1, b1, W2, b2, Wo, bo):
    B = user_ids.shape[0]
    D = user_table.shape[1]
    H1 = W1.shape[1]
    H2 = W2.shape[1]

    ue, ie = _gather_fn(B, D)(user_ids, user_table, item_ids, item_table)

    w1u = W1[:D]
    w1i = W1[D:]
    out = _mlp_fn(B, D, H1, H2, 512)(
        ue, ie, w1u, w1i, b1.reshape(1, H1), W2, b2.reshape(1, H2),
        Wo.reshape(1, H2), bo)
    return out
